# Initial kernel scaffold; baseline (speedup 1.0000x reference)
import jax
import jax.numpy as jnp
from jax.experimental import pallas as pl


def kernel(x, edge_index, gumbel_u, W_enc, b_enc, W1, b1, W2, b2):
    raise NotImplementedError("write your pallas kernel here")


# speedup vs baseline: 4.6591x; 4.6591x over previous
import functools

import jax
import jax.numpy as jnp
from jax import lax
from jax.experimental import pallas as pl
from jax.experimental.pallas import tpu as pltpu
from jax.experimental.pallas import tpu_sc as plsc

NNODES = 10000
NEDGES = 320000
D = 128
H = 128

NC = 2
NS = 16
NW = NC * NS
LANES = 16

EPT = NEDGES // NW
K = 80
NCH = EPT // K
NBLOCK = 5
CPB = NCH // NBLOCK
STRIPE = 632
LAST_STRIPE = NNODES - (NS - 1) * STRIPE

_MESH = plsc.VectorSubcoreMesh(
    core_axis_name="c", subcore_axis_name="s", num_cores=NC, num_subcores=NS)


@functools.partial(
    pl.kernel,
    mesh=_MESH,
    out_type=jax.ShapeDtypeStruct((NC * NNODES, D), jnp.float32),
    scratch_types=[
        pltpu.VMEM((EPT,), jnp.int32),
        pltpu.VMEM((NCH, K), jnp.int32),
        pltpu.VMEM((K, D), jnp.float32),
        pltpu.VMEM((K, D), jnp.float32),
        pltpu.VMEM_SHARED((NNODES, D), jnp.float32),
        pltpu.SemaphoreType.DMA,
        pltpu.SemaphoreType.DMA,
    ],
)
def _scatter_add(x_hbm, src_hbm, dst_hbm, out_hbm,
                 src_v, dst_v, r0, r1, acc_sh, sem0, sem1):
    c = lax.axis_index("c")
    s = lax.axis_index("s")
    wid = s * NC + c
    start = pl.multiple_of(s * STRIPE, 8)

    @pl.when(s < NS - 1)
    def _():
        pltpu.sync_copy(x_hbm.at[pl.ds(start, STRIPE)],
                        acc_sh.at[pl.ds(start, STRIPE)])

    @pl.when(s == NS - 1)
    def _():
        pltpu.sync_copy(x_hbm.at[pl.ds((NS - 1) * STRIPE, LAST_STRIPE)],
                        acc_sh.at[pl.ds((NS - 1) * STRIPE, LAST_STRIPE)])

    pltpu.sync_copy(src_hbm.at[wid], src_v)
    pltpu.sync_copy(dst_hbm.at[wid], dst_v)
    plsc.subcore_barrier()

    def sidx(ch):
        return src_v.at[pl.ds(pl.multiple_of(ch * K, 8), K)]

    pltpu.async_copy(x_hbm.at[sidx(0)], r0, sem0)
    pltpu.async_copy(x_hbm.at[sidx(1)], r1, sem1)

    def pair_body(i, carry):
        for b, (r, sm) in enumerate(((r0, sem0), (r1, sem1))):
            ch = i * 2 + b
            pltpu.make_async_copy(x_hbm.at[sidx(ch)], r, sm).wait()
            pltpu.sync_copy(r, acc_sh.at[dst_v.at[ch]], add=True)
            nxt = ch + 2

            @pl.when(nxt < NCH)
            def _():
                pltpu.async_copy(x_hbm.at[sidx(nxt)], r, sm)
        return carry

    lax.fori_loop(0, NCH // 2, pair_body, 0)
    pltpu.make_async_copy(x_hbm.at[sidx(NCH - 1)], r0, sem0).wait()
    pltpu.sync_copy(r0, acc_sh.at[dst_v.at[NCH - 1]], add=True)

    plsc.subcore_barrier()
    obase = pl.multiple_of(c * NNODES + s * STRIPE, 8)

    @pl.when(s < NS - 1)
    def _():
        pltpu.sync_copy(acc_sh.at[pl.ds(start, STRIPE)],
                        out_hbm.at[pl.ds(obase, STRIPE)])

    @pl.when(s == NS - 1)
    def _():
        pltpu.sync_copy(
            acc_sh.at[pl.ds((NS - 1) * STRIPE, LAST_STRIPE)],
            out_hbm.at[pl.ds(pl.multiple_of(c * NNODES + (NS - 1) * STRIPE, 8),
                             LAST_STRIPE)])


def _dense_body(x_ref, a0_ref, a1_ref, wenc_ref, benc_ref, w1t_ref, w1b_ref,
                b1_ref, pt_ref, pb_ref):
    sblk = a0_ref[...] + a1_ref[...] - x_ref[...]
    emb = jnp.dot(sblk.astype(jnp.bfloat16),
                  wenc_ref[...].astype(jnp.bfloat16),
                  preferred_element_type=jnp.float32)
    emb = jnp.maximum(emb + benc_ref[...], 0.0)
    emb_bf = emb.astype(jnp.bfloat16)
    pt_ref[...] = jnp.dot(emb_bf, w1t_ref[...].astype(jnp.bfloat16),
                          preferred_element_type=jnp.float32)
    pb_ref[...] = (jnp.dot(emb_bf, w1b_ref[...].astype(jnp.bfloat16),
                           preferred_element_type=jnp.float32)
                   + b1_ref[...])


_NBLK = 10
_BR = NNODES // _NBLK


def _dense_call(x, a0, a1, wenc, benc, w1t, w1b, b1):
    row = lambda i: (i, 0)
    full = lambda i: (0, 0)
    return pl.pallas_call(
        _dense_body,
        grid=(_NBLK,),
        in_specs=[
            pl.BlockSpec((_BR, D), row),
            pl.BlockSpec((_BR, D), row),
            pl.BlockSpec((_BR, D), row),
            pl.BlockSpec((D, D), full),
            pl.BlockSpec((1, D), full),
            pl.BlockSpec((D, H), full),
            pl.BlockSpec((D, H), full),
            pl.BlockSpec((1, H), full),
        ],
        out_specs=[
            pl.BlockSpec((_BR, H), row),
            pl.BlockSpec((_BR, H), row),
        ],
        out_shape=[
            jax.ShapeDtypeStruct((NNODES, H), jnp.float32),
            jax.ShapeDtypeStruct((NNODES, H), jnp.float32),
        ],
    )(x, a0, a1, wenc, benc, w1t, w1b, b1)


@functools.partial(
    pl.kernel,
    mesh=_MESH,
    out_type=jax.ShapeDtypeStruct((NW * EPT * LANES,), jnp.float32),
    scratch_types=[
        pltpu.VMEM((EPT,), jnp.int32),
        pltpu.VMEM((EPT,), jnp.int32),
        pltpu.VMEM((K, H), jnp.float32),
        pltpu.VMEM((K, H), jnp.float32),
        pltpu.VMEM((K, H), jnp.float32),
        pltpu.VMEM((K, H), jnp.float32),
        pltpu.VMEM((CPB * K * LANES,), jnp.float32),
        pltpu.VMEM((H // LANES, LANES), jnp.float32),
        pltpu.SemaphoreType.DMA,
        pltpu.SemaphoreType.DMA,
        pltpu.SemaphoreType.DMA,
        pltpu.SemaphoreType.DMA,
    ],
)
def _edge_score(pt_hbm, pb_hbm, src_hbm, dst_hbm, w2d_hbm, out_hbm,
                src_v, dst_v, ra0, rb0, ra1, rb1, oc_v, w2d_v,
                semA0, semB0, semA1, semB1):
    c = lax.axis_index("c")
    s = lax.axis_index("s")
    wid = s * NC + c
    base = wid * EPT
    pltpu.sync_copy(src_hbm.at[wid], src_v)
    pltpu.sync_copy(dst_hbm.at[wid], dst_v)
    pltpu.sync_copy(w2d_hbm, w2d_v)

    w2regs = [w2d_v[j] for j in range(H // LANES)]
    bufs = ((ra0, rb0, semA0, semB0), (ra1, rb1, semA1, semB1))

    def sidx(ch):
        return src_v.at[pl.ds(pl.multiple_of(ch * K, 8), K)]

    def didx(ch):
        return dst_v.at[pl.ds(pl.multiple_of(ch * K, 8), K)]

    def bf16_round(v):
        y = lax.bitcast_convert_type(v, jnp.int32)
        r = y + jnp.int32(0x7FFF) + ((y >> jnp.int32(16)) & jnp.int32(1))
        return lax.bitcast_convert_type(r & jnp.int32(-65536), jnp.float32)

    def compute_chunk(ra, rb, local_ch):
        def ebody(e, carry):
            acc = jnp.zeros((LANES,), jnp.float32)
            for j in range(H // LANES):
                av = ra[e, pl.ds(j * LANES, LANES)]
                bv = rb[e, pl.ds(j * LANES, LANES)]
                acc = acc + bf16_round(jnp.maximum(av + bv, 0.0)) * w2regs[j]
            off = pl.multiple_of((local_ch * K + e) * LANES, 8)
            oc_v[pl.ds(off, LANES)] = acc
            return carry

        lax.fori_loop(0, K, ebody, 0)

    def do_chunk(ch, local_ch, par):
        ra, rb, sA, sB = bufs[par]
        pltpu.async_copy(pt_hbm.at[sidx(ch)], ra, sA)
        pltpu.async_copy(pb_hbm.at[didx(ch)], rb, sB)
        pltpu.make_async_copy(pt_hbm.at[sidx(ch)], ra, sA).wait()
        pltpu.make_async_copy(pb_hbm.at[didx(ch)], rb, sB).wait()
        compute_chunk(ra, rb, local_ch)

    for blk in range(NBLOCK):
        par0 = (blk * CPB) % 2

        def pair_body(i, carry, blk=blk, par0=par0):
            for b in range(2):
                ch = blk * CPB + i * 2 + b
                do_chunk(ch, i * 2 + b, (par0 + b) % 2)
            return carry

        lax.fori_loop(0, CPB // 2, pair_body, 0)
        do_chunk(blk * CPB + CPB - 1, CPB - 1, (par0 + CPB - 1) % 2)
        pltpu.sync_copy(
            oc_v,
            out_hbm.at[pl.ds(pl.multiple_of((base + blk * CPB * K) * LANES, 8),
                             CPB * K * LANES)])


def _final_body(acc_ref, u_ref, sel_ref, gm_ref, b2_ref, out_ref):
    ssum = jnp.dot(acc_ref[...], sel_ref[...],
                   preferred_element_type=jnp.float32,
                   precision=lax.Precision.HIGHEST)
    g = -jnp.log(-jnp.log(u_ref[...]))
    gd = jnp.dot(g, gm_ref[...], preferred_element_type=jnp.float32,
                 precision=lax.Precision.HIGHEST)
    dec = ssum + (b2_ref[1] - b2_ref[0]) + gd
    out_ref[...] = jnp.where(dec > 0.0, 1.0, 0.0).astype(jnp.float32)


_EROWS = NEDGES // 128
_ECOLS = 128 * LANES


def _final_call(acc, u, sel, gm, b2):
    full = lambda: (0, 0)
    return pl.pallas_call(
        _final_body,
        in_specs=[
            pl.BlockSpec((_EROWS, _ECOLS), full),
            pl.BlockSpec((_EROWS, 256), full),
            pl.BlockSpec((_ECOLS, 128), full),
            pl.BlockSpec((256, 128), full),
            pl.BlockSpec(memory_space=pltpu.SMEM),
        ],
        out_specs=pl.BlockSpec((_EROWS, 128), full),
        out_shape=jax.ShapeDtypeStruct((_EROWS, 128), jnp.float32),
    )(acc, u, sel, gm, b2)


def kernel(x, edge_index, gumbel_u, W_enc, b_enc, W1, b1, W2, b2):
    src2 = edge_index[0].reshape(NW, EPT)
    dst2 = edge_index[1].reshape(NW, EPT)
    dst3 = edge_index[1].reshape(NW, NCH, K)

    agg2 = _scatter_add(x, src2, dst3)
    agg2 = lax.optimization_barrier(agg2)
    a0, a1 = agg2[:NNODES], agg2[NNODES:]

    pt, pb = _dense_call(x, a0, a1, W_enc, b_enc.reshape(1, D),
                         W1[:D], W1[D:], b1.reshape(1, H))
    pt, pb = lax.optimization_barrier((pt, pb))

    w2_bf = W2.astype(jnp.bfloat16).astype(jnp.float32)
    w28 = (w2_bf[:, 1] - w2_bf[:, 0]).reshape(H // LANES, LANES)
    acc = _edge_score(pt, pb, src2, dst2, w28)
    acc = lax.optimization_barrier(acc)

    cols = jnp.arange(128, dtype=jnp.int32)[None, :]
    sel = (jnp.arange(_ECOLS, dtype=jnp.int32)[:, None] // LANES
           == cols).astype(jnp.float32)
    rows = jnp.arange(256, dtype=jnp.int32)[:, None]
    gm = jnp.where(rows == 2 * cols + 1, 1.0,
                   jnp.where(rows == 2 * cols, -1.0, 0.0)
                   ).astype(jnp.float32)
    acc_r = acc.reshape(_EROWS, _ECOLS)
    u_r = gumbel_u.reshape(_EROWS, 256)
    out2d = _final_call(acc_r, u_r, sel, gm, b2)
    return out2d.reshape(NEDGES)


# speedup vs baseline: 4.7265x; 1.0145x over previous
import functools

import jax
import jax.numpy as jnp
from jax import lax
from jax.experimental import pallas as pl
from jax.experimental.pallas import tpu as pltpu
from jax.experimental.pallas import tpu_sc as plsc

NNODES = 10000
NEDGES = 320000
D = 128
H = 128

NC = 2
NS = 16
NW = NC * NS
LANES = 16

EPT = NEDGES // NW
K = 80
NCH = EPT // K
NBLOCK = 5
CPB = NCH // NBLOCK
STRIPE = 632
LAST_STRIPE = NNODES - (NS - 1) * STRIPE

_MESH = plsc.VectorSubcoreMesh(
    core_axis_name="c", subcore_axis_name="s", num_cores=NC, num_subcores=NS)


@functools.partial(
    pl.kernel,
    mesh=_MESH,
    out_type=jax.ShapeDtypeStruct((NC * NNODES, D), jnp.float32),
    scratch_types=[
        pltpu.VMEM((EPT,), jnp.int32),
        pltpu.VMEM((NCH, K), jnp.int32),
        pltpu.VMEM((K, D), jnp.float32),
        pltpu.VMEM((K, D), jnp.float32),
        pltpu.VMEM_SHARED((NNODES, D), jnp.float32),
        pltpu.SemaphoreType.DMA,
        pltpu.SemaphoreType.DMA,
    ],
)
def _scatter_add(x_hbm, src_hbm, dst_hbm, out_hbm,
                 src_v, dst_v, r0, r1, acc_sh, sem0, sem1):
    c = lax.axis_index("c")
    s = lax.axis_index("s")
    wid = s * NC + c
    start = pl.multiple_of(s * STRIPE, 8)

    @pl.when(s < NS - 1)
    def _():
        pltpu.sync_copy(x_hbm.at[pl.ds(start, STRIPE)],
                        acc_sh.at[pl.ds(start, STRIPE)])

    @pl.when(s == NS - 1)
    def _():
        pltpu.sync_copy(x_hbm.at[pl.ds((NS - 1) * STRIPE, LAST_STRIPE)],
                        acc_sh.at[pl.ds((NS - 1) * STRIPE, LAST_STRIPE)])

    pltpu.sync_copy(src_hbm.at[wid], src_v)
    pltpu.sync_copy(dst_hbm.at[wid], dst_v)
    plsc.subcore_barrier()

    def sidx(ch):
        return src_v.at[pl.ds(pl.multiple_of(ch * K, 8), K)]

    pltpu.async_copy(x_hbm.at[sidx(0)], r0, sem0)
    pltpu.async_copy(x_hbm.at[sidx(1)], r1, sem1)

    def pair_body(i, carry):
        for b, (r, sm) in enumerate(((r0, sem0), (r1, sem1))):
            ch = i * 2 + b
            pltpu.make_async_copy(x_hbm.at[sidx(ch)], r, sm).wait()
            pltpu.sync_copy(r, acc_sh.at[dst_v.at[ch]], add=True)
            nxt = ch + 2

            @pl.when(nxt < NCH)
            def _():
                pltpu.async_copy(x_hbm.at[sidx(nxt)], r, sm)
        return carry

    lax.fori_loop(0, NCH // 2, pair_body, 0)
    pltpu.make_async_copy(x_hbm.at[sidx(NCH - 1)], r0, sem0).wait()
    pltpu.sync_copy(r0, acc_sh.at[dst_v.at[NCH - 1]], add=True)

    plsc.subcore_barrier()
    obase = pl.multiple_of(c * NNODES + s * STRIPE, 8)

    @pl.when(s < NS - 1)
    def _():
        pltpu.sync_copy(acc_sh.at[pl.ds(start, STRIPE)],
                        out_hbm.at[pl.ds(obase, STRIPE)])

    @pl.when(s == NS - 1)
    def _():
        pltpu.sync_copy(
            acc_sh.at[pl.ds((NS - 1) * STRIPE, LAST_STRIPE)],
            out_hbm.at[pl.ds(pl.multiple_of(c * NNODES + (NS - 1) * STRIPE, 8),
                             LAST_STRIPE)])


def _dense_body(x_ref, a0_ref, a1_ref, wenc_ref, benc_ref, w1t_ref, w1b_ref,
                b1_ref, pt_ref, pb_ref):
    sblk = a0_ref[...] + a1_ref[...] - x_ref[...]
    emb = jnp.dot(sblk.astype(jnp.bfloat16),
                  wenc_ref[...].astype(jnp.bfloat16),
                  preferred_element_type=jnp.float32)
    emb = jnp.maximum(emb + benc_ref[...], 0.0)
    emb_bf = emb.astype(jnp.bfloat16)
    pt_ref[...] = jnp.dot(emb_bf, w1t_ref[...].astype(jnp.bfloat16),
                          preferred_element_type=jnp.float32)
    pb_ref[...] = (jnp.dot(emb_bf, w1b_ref[...].astype(jnp.bfloat16),
                           preferred_element_type=jnp.float32)
                   + b1_ref[...])


_NBLK = 10
_BR = NNODES // _NBLK


def _dense_call(x, agg2, wenc, benc, w1t, w1b, b1):
    row = lambda i: (i, 0)
    full = lambda i: (0, 0)
    return pl.pallas_call(
        _dense_body,
        grid=(_NBLK,),
        in_specs=[
            pl.BlockSpec((_BR, D), row),
            pl.BlockSpec((_BR, D), row),
            pl.BlockSpec((_BR, D), lambda i: (_NBLK + i, 0)),
            pl.BlockSpec((D, D), full),
            pl.BlockSpec((1, D), full),
            pl.BlockSpec((D, H), full),
            pl.BlockSpec((D, H), full),
            pl.BlockSpec((1, H), full),
        ],
        out_specs=[
            pl.BlockSpec((_BR, H), row),
            pl.BlockSpec((_BR, H), row),
        ],
        out_shape=[
            jax.ShapeDtypeStruct((NNODES, H), jnp.float32),
            jax.ShapeDtypeStruct((NNODES, H), jnp.float32),
        ],
    )(x, agg2, agg2, wenc, benc, w1t, w1b, b1)


@functools.partial(
    pl.kernel,
    mesh=_MESH,
    out_type=jax.ShapeDtypeStruct((NW * EPT * LANES,), jnp.float32),
    scratch_types=[
        pltpu.VMEM((EPT,), jnp.int32),
        pltpu.VMEM((EPT,), jnp.int32),
        pltpu.VMEM((K, H), jnp.float32),
        pltpu.VMEM((K, H), jnp.float32),
        pltpu.VMEM((K, H), jnp.float32),
        pltpu.VMEM((K, H), jnp.float32),
        pltpu.VMEM((CPB * K * LANES,), jnp.float32),
        pltpu.VMEM((H // LANES, LANES), jnp.float32),
        pltpu.SemaphoreType.DMA,
        pltpu.SemaphoreType.DMA,
        pltpu.SemaphoreType.DMA,
        pltpu.SemaphoreType.DMA,
    ],
)
def _edge_score(pt_hbm, pb_hbm, src_hbm, dst_hbm, w2d_hbm, out_hbm,
                src_v, dst_v, ra0, rb0, ra1, rb1, oc_v, w2d_v,
                semA0, semB0, semA1, semB1):
    c = lax.axis_index("c")
    s = lax.axis_index("s")
    wid = s * NC + c
    base = wid * EPT
    pltpu.sync_copy(src_hbm.at[wid], src_v)
    pltpu.sync_copy(dst_hbm.at[wid], dst_v)
    pltpu.sync_copy(w2d_hbm, w2d_v)

    w2regs = [w2d_v[j] for j in range(H // LANES)]
    bufs = ((ra0, rb0, semA0, semB0), (ra1, rb1, semA1, semB1))

    def sidx(ch):
        return src_v.at[pl.ds(pl.multiple_of(ch * K, 8), K)]

    def didx(ch):
        return dst_v.at[pl.ds(pl.multiple_of(ch * K, 8), K)]

    def bf16_round(v):
        y = lax.bitcast_convert_type(v, jnp.int32)
        r = y + jnp.int32(0x7FFF) + ((y >> jnp.int32(16)) & jnp.int32(1))
        return lax.bitcast_convert_type(r & jnp.int32(-65536), jnp.float32)

    def compute_chunk(ra, rb, local_ch):
        def ebody(e, carry):
            acc = jnp.zeros((LANES,), jnp.float32)
            for j in range(H // LANES):
                av = ra[e, pl.ds(j * LANES, LANES)]
                bv = rb[e, pl.ds(j * LANES, LANES)]
                acc = acc + bf16_round(jnp.maximum(av + bv, 0.0)) * w2regs[j]
            off = pl.multiple_of((local_ch * K + e) * LANES, 8)
            oc_v[pl.ds(off, LANES)] = acc
            return carry

        lax.fori_loop(0, K, ebody, 0)

    def do_chunk(ch, local_ch, par):
        ra, rb, sA, sB = bufs[par]
        pltpu.async_copy(pt_hbm.at[sidx(ch)], ra, sA)
        pltpu.async_copy(pb_hbm.at[didx(ch)], rb, sB)
        pltpu.make_async_copy(pt_hbm.at[sidx(ch)], ra, sA).wait()
        pltpu.make_async_copy(pb_hbm.at[didx(ch)], rb, sB).wait()
        compute_chunk(ra, rb, local_ch)

    for blk in range(NBLOCK):
        par0 = (blk * CPB) % 2

        def pair_body(i, carry, blk=blk, par0=par0):
            for b in range(2):
                ch = blk * CPB + i * 2 + b
                do_chunk(ch, i * 2 + b, (par0 + b) % 2)
            return carry

        lax.fori_loop(0, CPB // 2, pair_body, 0)
        do_chunk(blk * CPB + CPB - 1, CPB - 1, (par0 + CPB - 1) % 2)
        pltpu.sync_copy(
            oc_v,
            out_hbm.at[pl.ds(pl.multiple_of((base + blk * CPB * K) * LANES, 8),
                             CPB * K * LANES)])


def _final_body(acc_ref, u_ref, sel_ref, gm_ref, b2_ref, out_ref):
    ssum = jnp.dot(acc_ref[...], sel_ref[...],
                   preferred_element_type=jnp.float32,
                   precision=lax.Precision.HIGHEST)
    g = -jnp.log(-jnp.log(u_ref[...]))
    gd = jnp.dot(g, gm_ref[...], preferred_element_type=jnp.float32,
                 precision=lax.Precision.HIGHEST)
    dec = ssum + (b2_ref[1] - b2_ref[0]) + gd
    out_ref[...] = jnp.where(dec > 0.0, 1.0, 0.0).astype(jnp.float32)


_EROWS = NEDGES // 128
_ECOLS = 128 * LANES


def _final_call(acc, u, sel, gm, b2):
    full = lambda: (0, 0)
    return pl.pallas_call(
        _final_body,
        in_specs=[
            pl.BlockSpec((_EROWS, _ECOLS), full),
            pl.BlockSpec((_EROWS, 256), full),
            pl.BlockSpec((_ECOLS, 128), full),
            pl.BlockSpec((256, 128), full),
            pl.BlockSpec(memory_space=pltpu.SMEM),
        ],
        out_specs=pl.BlockSpec((_EROWS, 128), full),
        out_shape=jax.ShapeDtypeStruct((_EROWS, 128), jnp.float32),
    )(acc, u, sel, gm, b2)


def kernel(x, edge_index, gumbel_u, W_enc, b_enc, W1, b1, W2, b2):
    src2 = edge_index[0].reshape(NW, EPT)
    dst2 = edge_index[1].reshape(NW, EPT)
    dst3 = edge_index[1].reshape(NW, NCH, K)

    agg2 = _scatter_add(x, src2, dst3)

    pt, pb = _dense_call(x, agg2, W_enc, b_enc.reshape(1, D),
                         W1[:D], W1[D:], b1.reshape(1, H))

    w2_bf = W2.astype(jnp.bfloat16).astype(jnp.float32)
    w28 = (w2_bf[:, 1] - w2_bf[:, 0]).reshape(H // LANES, LANES)
    acc = _edge_score(pt, pb, src2, dst2, w28)

    cols = jnp.arange(128, dtype=jnp.int32)[None, :]
    sel = (jnp.arange(_ECOLS, dtype=jnp.int32)[:, None] // LANES
           == cols).astype(jnp.float32)
    rows = jnp.arange(256, dtype=jnp.int32)[:, None]
    gm = jnp.where(rows == 2 * cols + 1, 1.0,
                   jnp.where(rows == 2 * cols, -1.0, 0.0)
                   ).astype(jnp.float32)
    acc_r = acc.reshape(_EROWS, _ECOLS)
    u_r = gumbel_u.reshape(_EROWS, 256)
    out2d = _final_call(acc_r, u_r, sel, gm, b2)
    return out2d.reshape(NEDGES)
